# baseline (speedup 1.0000x reference)
import functools

import jax
import jax.numpy as jnp
from jax.experimental import pallas as pl
from jax.experimental.pallas import tpu as pltpu

N = 10000
_BR1 = 160
_BR2 = 320
_BP = 2000


def _proj_body(x_ref, w_ref, as_ref, an_ref,
               h_ref, us_ref, us2_ref, vn_ref, vn2_ref):
    h = jnp.dot(x_ref[...], w_ref[...], preferred_element_type=jnp.float32)
    h_ref[...] = h
    fs = jnp.dot(h, as_ref[...], preferred_element_type=jnp.float32)
    fn = jnp.dot(h, an_ref[...], preferred_element_type=jnp.float32)
    us_ref[...] = jnp.exp(fs)
    us2_ref[...] = jnp.exp(0.2 * fs)
    vn_ref[...] = jnp.exp(fn)
    vn2_ref[...] = jnp.exp(0.2 * fn)


def _project(x, w, a_s, a_n):
    n, f = x.shape
    c = w.shape[1]
    grid = (n // _BP,)
    colspec = pl.BlockSpec((_BP, 1), lambda i: (i, 0))
    colshape = jax.ShapeDtypeStruct((n, 1), jnp.float32)
    return pl.pallas_call(
        _proj_body,
        grid=grid,
        in_specs=[
            pl.BlockSpec((_BP, f), lambda i: (i, 0)),
            pl.BlockSpec((f, c), lambda i: (0, 0)),
            pl.BlockSpec((c, 1), lambda i: (0, 0)),
            pl.BlockSpec((c, 1), lambda i: (0, 0)),
        ],
        out_specs=[pl.BlockSpec((_BP, c), lambda i: (i, 0)),
                   colspec, colspec, colspec, colspec],
        out_shape=[jax.ShapeDtypeStruct((n, c), jnp.float32),
                   colshape, colshape, colshape, colshape],
    )(x, w, a_s, a_n)


def _attend(us, us2, vn, vn2, pm, h, b, c, final_softmax):
    p = jnp.maximum(us * vn, us2 * vn2)
    p = jnp.where(pm, p, 0.0)
    a = jnp.dot(p.astype(jnp.bfloat16), h,
                preferred_element_type=jnp.float32)
    num = a[:, :c]
    den = a[:, c:c + 1]
    den = jnp.where(den > 0.0, den, 1.0)
    out = num / den + b
    if final_softmax:
        mm = jnp.max(out, axis=-1, keepdims=True)
        ex = jnp.exp(out - mm)
        return ex / jnp.sum(ex, axis=-1, keepdims=True)
    return jnp.maximum(out, 0.0)


def _flash1_body(us_ref, us2_ref, vn_ref, vn2_ref, adj_ref, h_ref, b_ref,
                 o_ref, adj8_ref, *, c):
    adj = adj_ref[...]
    adj8_ref[...] = adj.astype(jnp.int8)
    o_ref[...] = _attend(us_ref[...], us2_ref[...], vn_ref[...],
                         vn2_ref[...], adj > 0.5, h_ref[...], b_ref[...],
                         c, final_softmax=False)


def _flash2_body(us_ref, us2_ref, vn_ref, vn2_ref, adj8_ref, h_ref, b_ref,
                 o_ref, *, c):
    o_ref[...] = _attend(us_ref[...], us2_ref[...], vn_ref[...],
                         vn2_ref[...], adj8_ref[...] != 0, h_ref[...],
                         b_ref[...], c, final_softmax=True)


def _flash_layer1(us, us2, vn, vn2, adj, h_aug, b):
    c1 = h_aug.shape[1]
    c = c1 - 1
    nr = pl.cdiv(N, _BR1)
    body = functools.partial(_flash1_body, c=c)
    colspec = pl.BlockSpec((_BR1, 1), lambda i: (i, 0))
    rowspec = pl.BlockSpec((1, N), lambda i: (0, 0))
    return pl.pallas_call(
        body,
        grid=(nr,),
        in_specs=[
            colspec, colspec, rowspec, rowspec,
            pl.BlockSpec((_BR1, N), lambda i: (i, 0)),
            pl.BlockSpec((N, c1), lambda i: (0, 0)),
            pl.BlockSpec((1, c), lambda i: (0, 0)),
        ],
        out_specs=[pl.BlockSpec((_BR1, c), lambda i: (i, 0)),
                   pl.BlockSpec((_BR1, N), lambda i: (i, 0))],
        out_shape=[jax.ShapeDtypeStruct((N, c), jnp.float32),
                   jax.ShapeDtypeStruct((N, N), jnp.int8)],
        compiler_params=pltpu.CompilerParams(
            dimension_semantics=("arbitrary",),
        ),
    )(us, us2, vn, vn2, adj, h_aug, b)


def _flash_layer2(us, us2, vn, vn2, adj8, h_aug, b):
    c1 = h_aug.shape[1]
    c = c1 - 1
    nr = pl.cdiv(N, _BR2)
    body = functools.partial(_flash2_body, c=c)
    colspec = pl.BlockSpec((_BR2, 1), lambda i: (i, 0))
    rowspec = pl.BlockSpec((1, N), lambda i: (0, 0))
    return pl.pallas_call(
        body,
        grid=(nr,),
        in_specs=[
            colspec, colspec, rowspec, rowspec,
            pl.BlockSpec((_BR2, N), lambda i: (i, 0)),
            pl.BlockSpec((N, c1), lambda i: (0, 0)),
            pl.BlockSpec((1, c), lambda i: (0, 0)),
        ],
        out_specs=pl.BlockSpec((_BR2, c), lambda i: (i, 0)),
        out_shape=jax.ShapeDtypeStruct((N, c), jnp.float32),
        compiler_params=pltpu.CompilerParams(
            dimension_semantics=("arbitrary",),
        ),
    )(us, us2, vn, vn2, adj8, h_aug, b)


def _augment(h):
    ones = jnp.ones((h.shape[0], 1), jnp.float32)
    return jnp.concatenate([h, ones], axis=1).astype(jnp.bfloat16)


def kernel(feats, adj, W1, a_self1, a_neigh1, b1, W2, a_self2, a_neigh2, b2):
    h1, us1, us21, vn1, vn21 = _project(feats, W1, a_self1, a_neigh1)
    x1, adj8 = _flash_layer1(us1, us21, vn1.reshape(1, N), vn21.reshape(1, N),
                             adj, _augment(h1), b1.reshape(1, -1))
    h2, us2_, us22, vn2_, vn22 = _project(x1, W2, a_self2, a_neigh2)
    out = _flash_layer2(us2_, us22, vn2_.reshape(1, N), vn22.reshape(1, N),
                        adj8, _augment(h2), b2.reshape(1, -1))
    return out


# speedup vs baseline: 1.1544x; 1.1544x over previous
import functools

import jax
import jax.numpy as jnp
from jax.experimental import pallas as pl
from jax.experimental.pallas import tpu as pltpu

N = 10000
_BR1 = 224
_BR2 = 320
_BP = 2000


def _proj_body(x_ref, w_ref, as_ref, an_ref,
               h_ref, us_ref, us2_ref, vn_ref, vn2_ref):
    h = jnp.dot(x_ref[...], w_ref[...], preferred_element_type=jnp.float32)
    h_ref[...] = h
    fs = jnp.dot(h, as_ref[...], preferred_element_type=jnp.float32)
    fn = jnp.dot(h, an_ref[...], preferred_element_type=jnp.float32)
    us_ref[...] = jnp.exp(fs).astype(jnp.bfloat16)
    us2_ref[...] = jnp.exp(0.2 * fs).astype(jnp.bfloat16)
    vn_ref[...] = jnp.exp(fn).astype(jnp.bfloat16)
    vn2_ref[...] = jnp.exp(0.2 * fn).astype(jnp.bfloat16)


def _project(x, w, a_s, a_n):
    n, f = x.shape
    c = w.shape[1]
    grid = (n // _BP,)
    colspec = pl.BlockSpec((_BP, 1), lambda i: (i, 0))
    colshape = jax.ShapeDtypeStruct((n, 1), jnp.bfloat16)
    return pl.pallas_call(
        _proj_body,
        grid=grid,
        in_specs=[
            pl.BlockSpec((_BP, f), lambda i: (i, 0)),
            pl.BlockSpec((f, c), lambda i: (0, 0)),
            pl.BlockSpec((c, 1), lambda i: (0, 0)),
            pl.BlockSpec((c, 1), lambda i: (0, 0)),
        ],
        out_specs=[pl.BlockSpec((_BP, c), lambda i: (i, 0)),
                   colspec, colspec, colspec, colspec],
        out_shape=[jax.ShapeDtypeStruct((n, c), jnp.float32),
                   colshape, colshape, colshape, colshape],
    )(x, w, a_s, a_n)


def _attend(us, us2, vn, vn2, pm, h, b, c, final_softmax):
    p = jnp.maximum(us * vn, us2 * vn2)
    p = p * pm
    a = jnp.dot(p, h, preferred_element_type=jnp.float32)
    num = a[:, :c]
    den = a[:, c:c + 1]
    den = jnp.where(den > 0.0, den, 1.0)
    out = num / den + b
    if final_softmax:
        mm = jnp.max(out, axis=-1, keepdims=True)
        ex = jnp.exp(out - mm)
        return ex / jnp.sum(ex, axis=-1, keepdims=True)
    return jnp.maximum(out, 0.0)


def _flash1_body(us_ref, us2_ref, vn_ref, vn2_ref, adj_ref, h_ref, b_ref,
                 o_ref, adj8_ref, *, c):
    adj = adj_ref[...]
    adj8_ref[...] = adj.astype(jnp.int8)
    o_ref[...] = _attend(us_ref[...], us2_ref[...], vn_ref[...],
                         vn2_ref[...], adj.astype(jnp.bfloat16),
                         h_ref[...], b_ref[...], c, final_softmax=False)


def _flash2_body(us_ref, us2_ref, vn_ref, vn2_ref, adj8_ref, h_ref, b_ref,
                 o_ref, *, c):
    o_ref[...] = _attend(us_ref[...], us2_ref[...], vn_ref[...],
                         vn2_ref[...], adj8_ref[...].astype(jnp.bfloat16),
                         h_ref[...], b_ref[...], c, final_softmax=True)


def _flash_layer1(us, us2, vn, vn2, adj, h_aug, b):
    c1 = h_aug.shape[1]
    c = c1 - 1
    nr = pl.cdiv(N, _BR1)
    body = functools.partial(_flash1_body, c=c)
    colspec = pl.BlockSpec((_BR1, 1), lambda i: (i, 0))
    rowspec = pl.BlockSpec((1, N), lambda i: (0, 0))
    return pl.pallas_call(
        body,
        grid=(nr,),
        in_specs=[
            colspec, colspec, rowspec, rowspec,
            pl.BlockSpec((_BR1, N), lambda i: (i, 0)),
            pl.BlockSpec((N, c1), lambda i: (0, 0)),
            pl.BlockSpec((1, c), lambda i: (0, 0)),
        ],
        out_specs=[pl.BlockSpec((_BR1, c), lambda i: (i, 0)),
                   pl.BlockSpec((_BR1, N), lambda i: (i, 0))],
        out_shape=[jax.ShapeDtypeStruct((N, c), jnp.float32),
                   jax.ShapeDtypeStruct((N, N), jnp.int8)],
        compiler_params=pltpu.CompilerParams(
            dimension_semantics=("arbitrary",),
        ),
    )(us, us2, vn, vn2, adj, h_aug, b)


def _flash_layer2(us, us2, vn, vn2, adj8, h_aug, b):
    c1 = h_aug.shape[1]
    c = c1 - 1
    nr = pl.cdiv(N, _BR2)
    body = functools.partial(_flash2_body, c=c)
    colspec = pl.BlockSpec((_BR2, 1), lambda i: (i, 0))
    rowspec = pl.BlockSpec((1, N), lambda i: (0, 0))
    return pl.pallas_call(
        body,
        grid=(nr,),
        in_specs=[
            colspec, colspec, rowspec, rowspec,
            pl.BlockSpec((_BR2, N), lambda i: (i, 0)),
            pl.BlockSpec((N, c1), lambda i: (0, 0)),
            pl.BlockSpec((1, c), lambda i: (0, 0)),
        ],
        out_specs=pl.BlockSpec((_BR2, c), lambda i: (i, 0)),
        out_shape=jax.ShapeDtypeStruct((N, c), jnp.float32),
        compiler_params=pltpu.CompilerParams(
            dimension_semantics=("arbitrary",),
        ),
    )(us, us2, vn, vn2, adj8, h_aug, b)


def _augment(h):
    ones = jnp.ones((h.shape[0], 1), jnp.float32)
    return jnp.concatenate([h, ones], axis=1).astype(jnp.bfloat16)


def kernel(feats, adj, W1, a_self1, a_neigh1, b1, W2, a_self2, a_neigh2, b2):
    h1, us1, us21, vn1, vn21 = _project(feats, W1, a_self1, a_neigh1)
    x1, adj8 = _flash_layer1(us1, us21, vn1.reshape(1, N), vn21.reshape(1, N),
                             adj, _augment(h1), b1.reshape(1, -1))
    h2, us2_, us22, vn2_, vn22 = _project(x1, W2, a_self2, a_neigh2)
    out = _flash_layer2(us2_, us22, vn2_.reshape(1, N), vn22.reshape(1, N),
                        adj8, _augment(h2), b2.reshape(1, -1))
    return out


# speedup vs baseline: 1.1850x; 1.0266x over previous
import functools

import jax
import jax.numpy as jnp
from jax.experimental import pallas as pl
from jax.experimental.pallas import tpu as pltpu

N = 10000
_BR1 = 320
_BR2 = 320
_BP = 2000


def _proj_body(x_ref, w_ref, as_ref, an_ref,
               h_ref, us_ref, us2_ref, vn_ref, vn2_ref):
    h = jnp.dot(x_ref[...], w_ref[...], preferred_element_type=jnp.float32)
    h_ref[...] = h
    fs = jnp.dot(h, as_ref[...], preferred_element_type=jnp.float32)
    fn = jnp.dot(h, an_ref[...], preferred_element_type=jnp.float32)
    us_ref[...] = jnp.exp(fs).astype(jnp.bfloat16)
    us2_ref[...] = jnp.exp(0.2 * fs).astype(jnp.bfloat16)
    vn_ref[...] = jnp.exp(fn).astype(jnp.bfloat16)
    vn2_ref[...] = jnp.exp(0.2 * fn).astype(jnp.bfloat16)


def _project(x, w, a_s, a_n):
    n, f = x.shape
    c = w.shape[1]
    grid = (n // _BP,)
    colspec = pl.BlockSpec((_BP, 1), lambda i: (i, 0))
    colshape = jax.ShapeDtypeStruct((n, 1), jnp.bfloat16)
    return pl.pallas_call(
        _proj_body,
        grid=grid,
        in_specs=[
            pl.BlockSpec((_BP, f), lambda i: (i, 0)),
            pl.BlockSpec((f, c), lambda i: (0, 0)),
            pl.BlockSpec((c, 1), lambda i: (0, 0)),
            pl.BlockSpec((c, 1), lambda i: (0, 0)),
        ],
        out_specs=[pl.BlockSpec((_BP, c), lambda i: (i, 0)),
                   colspec, colspec, colspec, colspec],
        out_shape=[jax.ShapeDtypeStruct((n, c), jnp.float32),
                   colshape, colshape, colshape, colshape],
    )(x, w, a_s, a_n)


def _attend(us, us2, vn, vn2, pm, h, b, c, final_softmax):
    p = jnp.maximum(us * vn, us2 * vn2)
    p = p * pm
    a = jnp.dot(p, h, preferred_element_type=jnp.float32)
    num = a[:, :c]
    den = a[:, c:c + 1]
    den = jnp.where(den > 0.0, den, 1.0)
    out = num / den + b
    if final_softmax:
        mm = jnp.max(out, axis=-1, keepdims=True)
        ex = jnp.exp(out - mm)
        return ex / jnp.sum(ex, axis=-1, keepdims=True)
    return jnp.maximum(out, 0.0)


def _flash1_body(us_ref, us2_ref, vn_ref, vn2_ref, adj_ref, h_ref, b_ref,
                 o_ref, adj8_ref, *, c):
    adj = adj_ref[...]
    adj8_ref[...] = adj.astype(jnp.int8)
    o_ref[...] = _attend(us_ref[...], us2_ref[...], vn_ref[...],
                         vn2_ref[...], adj.astype(jnp.bfloat16),
                         h_ref[...], b_ref[...], c, final_softmax=False)


def _flash2_body(us_ref, us2_ref, vn_ref, vn2_ref, adj8_ref, h_ref, b_ref,
                 o_ref, *, c):
    o_ref[...] = _attend(us_ref[...], us2_ref[...], vn_ref[...],
                         vn2_ref[...], adj8_ref[...].astype(jnp.bfloat16),
                         h_ref[...], b_ref[...], c, final_softmax=True)


def _flash_layer1(us, us2, vn, vn2, adj, h_aug, b):
    c1 = h_aug.shape[1]
    c = c1 - 1
    nr = pl.cdiv(N, _BR1)
    body = functools.partial(_flash1_body, c=c)
    colspec = pl.BlockSpec((_BR1, 1), lambda i: (i, 0))
    rowspec = pl.BlockSpec((1, N), lambda i: (0, 0))
    return pl.pallas_call(
        body,
        grid=(nr,),
        in_specs=[
            colspec, colspec, rowspec, rowspec,
            pl.BlockSpec((_BR1, N), lambda i: (i, 0)),
            pl.BlockSpec((N, c1), lambda i: (0, 0)),
            pl.BlockSpec((1, c), lambda i: (0, 0)),
        ],
        out_specs=[pl.BlockSpec((_BR1, c), lambda i: (i, 0)),
                   pl.BlockSpec((_BR1, N), lambda i: (i, 0))],
        out_shape=[jax.ShapeDtypeStruct((N, c), jnp.float32),
                   jax.ShapeDtypeStruct((N, N), jnp.int8)],
        compiler_params=pltpu.CompilerParams(
            dimension_semantics=("arbitrary",),
        ),
    )(us, us2, vn, vn2, adj, h_aug, b)


def _flash_layer2(us, us2, vn, vn2, adj8, h_aug, b):
    c1 = h_aug.shape[1]
    c = c1 - 1
    nr = pl.cdiv(N, _BR2)
    body = functools.partial(_flash2_body, c=c)
    colspec = pl.BlockSpec((_BR2, 1), lambda i: (i, 0))
    rowspec = pl.BlockSpec((1, N), lambda i: (0, 0))
    return pl.pallas_call(
        body,
        grid=(nr,),
        in_specs=[
            colspec, colspec, rowspec, rowspec,
            pl.BlockSpec((_BR2, N), lambda i: (i, 0)),
            pl.BlockSpec((N, c1), lambda i: (0, 0)),
            pl.BlockSpec((1, c), lambda i: (0, 0)),
        ],
        out_specs=pl.BlockSpec((_BR2, c), lambda i: (i, 0)),
        out_shape=jax.ShapeDtypeStruct((N, c), jnp.float32),
        compiler_params=pltpu.CompilerParams(
            dimension_semantics=("arbitrary",),
        ),
    )(us, us2, vn, vn2, adj8, h_aug, b)


def _augment(h):
    ones = jnp.ones((h.shape[0], 1), jnp.float32)
    return jnp.concatenate([h, ones], axis=1).astype(jnp.bfloat16)


def kernel(feats, adj, W1, a_self1, a_neigh1, b1, W2, a_self2, a_neigh2, b2):
    h1, us1, us21, vn1, vn21 = _project(feats, W1, a_self1, a_neigh1)
    x1, adj8 = _flash_layer1(us1, us21, vn1.reshape(1, N), vn21.reshape(1, N),
                             adj, _augment(h1), b1.reshape(1, -1))
    h2, us2_, us22, vn2_, vn22 = _project(x1, W2, a_self2, a_neigh2)
    out = _flash_layer2(us2_, us22, vn2_.reshape(1, N), vn22.reshape(1, N),
                        adj8, _augment(h2), b2.reshape(1, -1))
    return out
